# baseline (speedup 1.0000x reference)
import functools

import jax
import jax.numpy as jnp
from jax import lax
from jax.experimental import pallas as pl
from jax.experimental.pallas import tpu as pltpu
from jax.experimental.pallas import tpu_sc as plsc

_NUM_PT = 32


def _transpose_table(rules):
    num_pt, v = rules.shape
    c = 4096

    def body(r_ref, o_ref):
        o_ref[...] = r_ref[...].T

    return pl.pallas_call(
        body,
        grid=(pl.cdiv(v, c),),
        in_specs=[pl.BlockSpec((num_pt, c), lambda i: (0, i))],
        out_specs=pl.BlockSpec((c, num_pt), lambda i: (i, 0)),
        out_shape=jax.ShapeDtypeStruct((v, num_pt), jnp.float32),
    )(rules)


def _sc_gather(table_t, idx_flat):
    n = idx_flat.shape[0]
    info = plsc.get_sparse_core_info()
    nc, ns = info.num_cores, info.num_subcores
    nw = nc * ns
    per_w = n // nw
    c = 800
    n_chunks = per_w // c
    mesh = plsc.VectorSubcoreMesh(core_axis_name="c", subcore_axis_name="s")

    @functools.partial(
        pl.kernel,
        mesh=mesh,
        out_type=jax.ShapeDtypeStruct((n, _NUM_PT), jnp.float32),
        scratch_types=[
            pltpu.VMEM((c,), jnp.int32),
            pltpu.VMEM((c, _NUM_PT), jnp.float32),
            pltpu.SemaphoreType.DMA,
        ],
        compiler_params=pltpu.CompilerParams(use_tc_tiling_on_sc=False),
    )
    def k(table_hbm, idx_hbm, out_hbm, idx_v, rows_v, sem):
        wid = lax.axis_index("s") * nc + lax.axis_index("c")

        def chunk_body(ci, carry):
            base = wid * per_w + ci * c
            pltpu.sync_copy(idx_hbm.at[pl.ds(base, c)], idx_v)
            pltpu.async_copy(table_hbm.at[idx_v], rows_v, sem).wait()
            pltpu.sync_copy(rows_v, out_hbm.at[pl.ds(base, c)])
            return carry

        lax.fori_loop(0, n_chunks, chunk_body, 0)

    return k(table_t, idx_flat)


def _transpose_out(rows, b, l):
    x = rows.reshape(b, l, _NUM_PT)
    cb = 8

    def body(i_ref, o_ref):
        o_ref[...] = jnp.transpose(i_ref[...], (0, 2, 1))

    return pl.pallas_call(
        body,
        grid=(b // cb,),
        in_specs=[pl.BlockSpec((cb, l, _NUM_PT), lambda i: (i, 0, 0))],
        out_specs=pl.BlockSpec((cb, _NUM_PT, l), lambda i: (i, 0, 0)),
        out_shape=jax.ShapeDtypeStruct((b, _NUM_PT, l), jnp.float32),
    )(x)


def kernel(sentences, rules):
    b, l = sentences.shape
    table_t = _transpose_table(rules)
    rows = _sc_gather(table_t, sentences.reshape(-1))
    return _transpose_out(rows, b, l)
